# baseline (speedup 1.0000x reference)
import functools

import jax
import jax.numpy as jnp
from jax import lax
from jax.experimental import pallas as pl
from jax.experimental.pallas import tpu as pltpu
from jax.experimental.pallas import tpu_sc as plsc

N_NODES = 10000
H = 128

NC = 2
NS = 16
CH = H // NC
GROUP = 128
NBUF = 4
NCHUNK = 4
R = 10240
RPT = R // NS


def _mlp_body(x_ref, w1, b1, w2, b2, w3, b3, w4, b4, o_ref):
    h = jax.nn.relu(jnp.dot(x_ref[...], w1[...],
                            preferred_element_type=jnp.float32) + b1[...])
    h = jax.nn.relu(jnp.dot(h, w2[...],
                            preferred_element_type=jnp.float32) + b2[...])
    h = jax.nn.relu(jnp.dot(h, w3[...],
                            preferred_element_type=jnp.float32) + b3[...]) + h
    h = jax.nn.relu(jnp.dot(h, w4[...],
                            preferred_element_type=jnp.float32) + b4[...]) + h
    o_ref[0] = h[:, :CH]
    o_ref[1] = h[:, CH:]


def _mlp(x_pad, W1, b1, W2, b2, W3, b3, W4, b4):
    blk = 1280
    grid = R // blk
    full = lambda a: pl.BlockSpec(a.shape, lambda i: (0,) * a.ndim)
    return pl.pallas_call(
        _mlp_body,
        grid=(grid,),
        in_specs=[
            pl.BlockSpec((blk, 5), lambda i: (i, 0)),
            full(W1), full(b1), full(W2), full(b2),
            full(W3), full(b3), full(W4), full(b4),
        ],
        out_specs=pl.BlockSpec((NC, blk, CH), lambda i: (0, i, 0)),
        out_shape=jax.ShapeDtypeStruct((NC, R, CH), jnp.float32),
    )(x_pad, W1, b1, W2, b2, W3, b3, W4, b4)


def _agg_body(n_groups, with_deg, h_hbm, src_hbm, dst_hbm, zrows_hbm,
              zdeg_hbm, acc_out, deg_out, sidx, didx, rows, ones_v, h_sh,
              acc_sh, deg_sh, gsems, ssems, dsem):
    cid = lax.axis_index("c")
    sid = lax.axis_index("s")
    row0 = sid * RPT
    gpc = n_groups // NCHUNK

    pltpu.sync_copy(zrows_hbm.at[pl.ds(row0, RPT)], acc_sh.at[pl.ds(row0, RPT)])
    pltpu.sync_copy(h_hbm.at[cid].at[pl.ds(row0, RPT)],
                    h_sh.at[pl.ds(row0, RPT)])
    if with_deg:
        pltpu.sync_copy(zdeg_hbm.at[pl.ds(row0, RPT)],
                        deg_sh.at[pl.ds(row0, RPT)])
        for j in range(GROUP // 16):
            ones_v[pl.ds(j * 16, 16)] = jnp.ones((16,), jnp.float32)
    plsc.subcore_barrier()

    def gather(g, b):
        src = h_hbm.at[cid] if b == 3 else h_sh
        return pltpu.async_copy(src.at[sidx.at[g]], rows.at[b], gsems.at[b])

    def gather_wait(g, b):
        src = h_hbm.at[cid] if b == 3 else h_sh
        pltpu.make_async_copy(src.at[sidx.at[g]], rows.at[b],
                              gsems.at[b]).wait()

    for c in range(NCHUNK):
        pltpu.sync_copy(src_hbm.at[sid, pl.ds(c * gpc, gpc)], sidx)
        pltpu.sync_copy(dst_hbm.at[sid, pl.ds(c * gpc, gpc)], didx)

        for b in range(NBUF):
            gather(b, b)

        def body(i, carry):
            g0 = i * NBUF
            scat = []
            for b in range(NBUF):
                g = g0 + b
                gather_wait(g, b)
                scat.append(pltpu.async_copy(rows.at[b],
                                             acc_sh.at[didx.at[g]],
                                             ssems.at[b], add=True))
                if with_deg:
                    scat.append(pltpu.async_copy(ones_v,
                                                 deg_sh.at[didx.at[g]],
                                                 dsem, add=True))
            for b in range(NBUF):
                g = g0 + b
                scat[(1 + with_deg) * b].wait()

                @pl.when(g + NBUF < gpc)
                def _():
                    gather(g + NBUF, b)

                if with_deg:
                    scat[(1 + with_deg) * b + 1].wait()
            return carry

        lax.fori_loop(0, gpc // NBUF, body, 0)

    plsc.subcore_barrier()
    pltpu.sync_copy(acc_sh.at[pl.ds(row0, RPT)],
                    acc_out.at[cid, pl.ds(row0, RPT)])
    if with_deg:
        pltpu.sync_copy(deg_sh.at[pl.ds(row0, RPT)],
                        deg_out.at[cid, pl.ds(row0, RPT)])


def _aggregate(h_split, src3, dst3, zrows, zdeg, n_groups, with_deg):
    mesh = plsc.VectorSubcoreMesh(core_axis_name="c", subcore_axis_name="s",
                                  num_cores=NC, num_subcores=NS)
    gpc = n_groups // NCHUNK
    k = pl.kernel(
        functools.partial(_agg_body, n_groups, with_deg),
        out_type=(jax.ShapeDtypeStruct((NC, R, CH), jnp.float32),
                  jax.ShapeDtypeStruct((NC, R), jnp.float32)),
        mesh=mesh,
        compiler_params=pltpu.CompilerParams(use_tc_tiling_on_sc=False),
        scratch_types=[
            pltpu.VMEM((gpc, GROUP), jnp.int32),
            pltpu.VMEM((gpc, GROUP), jnp.int32),
            pltpu.VMEM((NBUF, GROUP, CH), jnp.float32),
            pltpu.VMEM((GROUP,), jnp.float32),
            pltpu.VMEM_SHARED((R, CH), jnp.float32),
            pltpu.VMEM_SHARED((R, CH), jnp.float32),
            pltpu.VMEM_SHARED((R,), jnp.float32),
            pltpu.SemaphoreType.DMA((NBUF,)),
            pltpu.SemaphoreType.DMA((NBUF,)),
            pltpu.SemaphoreType.DMA,
        ],
    )
    return k(h_split, src3, dst3, zrows, zdeg)


def _conv_body(acc_ref, deg_ref, wc, bc, o_ref):
    s = jnp.concatenate([acc_ref[0], acc_ref[1]], axis=1)
    deg = jnp.maximum(deg_ref[0], 1.0)
    mean = s / deg[:, None]
    h = jax.nn.relu(
        jnp.dot(mean, wc[...], preferred_element_type=jnp.float32) + bc[...])
    o_ref[0] = h[:, :CH]
    o_ref[1] = h[:, CH:]


def _conv(acc, deg, Wc, bc):
    blk = 1280
    grid = R // blk
    full = lambda a: pl.BlockSpec(a.shape, lambda i: (0,) * a.ndim)
    return pl.pallas_call(
        _conv_body,
        grid=(grid,),
        in_specs=[
            pl.BlockSpec((NC, blk, CH), lambda i: (0, i, 0)),
            pl.BlockSpec((NC, blk), lambda i: (0, i)),
            full(Wc), full(bc),
        ],
        out_specs=pl.BlockSpec((NC, blk, CH), lambda i: (0, i, 0)),
        out_shape=jax.ShapeDtypeStruct((NC, R, CH), jnp.float32),
    )(acc, deg, Wc, bc)


def _conv_final_body(acc_ref, deg_ref, wc, bc, w5, b5, o_ref):
    s = jnp.concatenate([acc_ref[0], acc_ref[1]], axis=1)
    deg = jnp.maximum(deg_ref[0], 1.0)
    mean = s / deg[:, None]
    h = jax.nn.relu(
        jnp.dot(mean, wc[...], preferred_element_type=jnp.float32) + bc[...])
    o_ref[...] = jnp.dot(h, w5[...], preferred_element_type=jnp.float32) + b5[...]


def _conv_final(acc, deg, Wc, bc, W5p, b5p):
    blk = 1280
    grid = R // blk
    full = lambda a: pl.BlockSpec(a.shape, lambda i: (0,) * a.ndim)
    return pl.pallas_call(
        _conv_final_body,
        grid=(grid,),
        in_specs=[
            pl.BlockSpec((NC, blk, CH), lambda i: (0, i, 0)),
            pl.BlockSpec((NC, blk), lambda i: (0, i)),
            full(Wc), full(bc), full(W5p), full(b5p),
        ],
        out_specs=pl.BlockSpec((blk, 64), lambda i: (i, 0)),
        out_shape=jax.ShapeDtypeStruct((R, 64), jnp.float32),
    )(acc, deg, Wc, bc, W5p, b5p)


def kernel(x, edge_index, W1, b1, W2, b2, W3, b3, W4, b4,
           Wc1, bc1, Wc2, bc2, W5, b5):
    E = edge_index.shape[1]
    n_groups = -(-E // (NS * GROUP))
    n_groups = -(-n_groups // (NBUF * NCHUNK)) * (NBUF * NCHUNK)
    e_pad = NS * n_groups * GROUP

    src3 = (jnp.full((e_pad,), 0, jnp.int32).at[:E].set(edge_index[0])
            .reshape(NS, n_groups, GROUP))
    dst3 = (jnp.full((e_pad,), N_NODES, jnp.int32).at[:E].set(edge_index[1])
            .reshape(NS, n_groups, GROUP))
    x_pad = jnp.zeros((R, 5), jnp.float32).at[:N_NODES].set(x)
    zrows = jnp.zeros((R, CH), jnp.float32)
    zdeg = jnp.zeros((R,), jnp.float32)

    b1r, b2r = b1.reshape(1, -1), b2.reshape(1, -1)
    b3r, b4r = b3.reshape(1, -1), b4.reshape(1, -1)
    bc1r, bc2r = bc1.reshape(1, -1), bc2.reshape(1, -1)
    W5p = jnp.zeros((H, 64), jnp.float32).at[:, :60].set(W5)
    b5p = jnp.zeros((1, 64), jnp.float32).at[0, :60].set(b5)

    h = _mlp(x_pad, W1, b1r, W2, b2r, W3, b3r, W4, b4r)
    acc1, deg1 = _aggregate(h, src3, dst3, zrows, zdeg, n_groups, True)
    h2 = _conv(acc1, deg1, Wc1, bc1r)
    acc2, _ = _aggregate(h2, src3, dst3, zrows, zdeg, n_groups, False)
    out = _conv_final(acc2, deg1, Wc2, bc2r, W5p, b5p)
    return out[:N_NODES, :60]


# speedup vs baseline: 1.0451x; 1.0451x over previous
import functools

import jax
import jax.numpy as jnp
from jax import lax
from jax.experimental import pallas as pl
from jax.experimental.pallas import tpu as pltpu
from jax.experimental.pallas import tpu_sc as plsc

N_NODES = 10000
H = 128

NC = 2
NS = 16
CH = H // NC
GROUP = 128
NBUF = 4
NCHUNK = 4
R = 10240
RPT = R // NS


def _mlp_body(x_ref, w1, b1, w2, b2, w3, b3, w4, b4, o_ref):
    h = jax.nn.relu(jnp.dot(x_ref[...], w1[...],
                            preferred_element_type=jnp.float32) + b1[...])
    h = jax.nn.relu(jnp.dot(h, w2[...],
                            preferred_element_type=jnp.float32) + b2[...])
    h = jax.nn.relu(jnp.dot(h, w3[...],
                            preferred_element_type=jnp.float32) + b3[...]) + h
    h = jax.nn.relu(jnp.dot(h, w4[...],
                            preferred_element_type=jnp.float32) + b4[...]) + h
    o_ref[0] = h[:, :CH]
    o_ref[1] = h[:, CH:]


def _mlp(x_pad, W1, b1, W2, b2, W3, b3, W4, b4):
    blk = 1280
    grid = R // blk
    full = lambda a: pl.BlockSpec(a.shape, lambda i: (0,) * a.ndim)
    return pl.pallas_call(
        _mlp_body,
        grid=(grid,),
        in_specs=[
            pl.BlockSpec((blk, 5), lambda i: (i, 0)),
            full(W1), full(b1), full(W2), full(b2),
            full(W3), full(b3), full(W4), full(b4),
        ],
        out_specs=pl.BlockSpec((NC, blk, CH), lambda i: (0, i, 0)),
        out_shape=jax.ShapeDtypeStruct((NC, R, CH), jnp.float32),
    )(x_pad, W1, b1, W2, b2, W3, b3, W4, b4)


def _agg_body(n_groups, with_deg, h_hbm, src_hbm, dst_hbm, zrows_hbm,
              zdeg_hbm, acc_out, deg_out, sidx, didx, rows, ones_v, h_sh,
              acc_sh, deg_sh, gsems, ssems, dsem):
    cid = lax.axis_index("c")
    sid = lax.axis_index("s")
    row0 = sid * RPT
    gpc = n_groups // NCHUNK

    pltpu.sync_copy(zrows_hbm.at[pl.ds(row0, RPT)], acc_sh.at[pl.ds(row0, RPT)])
    pltpu.sync_copy(h_hbm.at[cid].at[pl.ds(row0, RPT)],
                    h_sh.at[pl.ds(row0, RPT)])
    if with_deg:
        pltpu.sync_copy(zdeg_hbm.at[pl.ds(row0, RPT)],
                        deg_sh.at[pl.ds(row0, RPT)])
        for j in range(GROUP // 16):
            ones_v[pl.ds(j * 16, 16)] = jnp.ones((16,), jnp.float32)
    plsc.subcore_barrier()

    def gather(g, b):
        return pltpu.async_copy(h_sh.at[sidx.at[g]], rows.at[b], gsems.at[b])

    def gather_wait(g, b):
        pltpu.make_async_copy(h_sh.at[sidx.at[g]], rows.at[b],
                              gsems.at[b]).wait()

    for c in range(NCHUNK):
        pltpu.sync_copy(src_hbm.at[sid, pl.ds(c * gpc, gpc)], sidx)
        pltpu.sync_copy(dst_hbm.at[sid, pl.ds(c * gpc, gpc)], didx)

        for b in range(NBUF):
            gather(b, b)

        def body(i, carry):
            g0 = i * NBUF
            scat = []
            for b in range(NBUF):
                g = g0 + b
                gather_wait(g, b)
                scat.append(pltpu.async_copy(rows.at[b],
                                             acc_sh.at[didx.at[g]],
                                             ssems.at[b], add=True))
                if with_deg:
                    scat.append(pltpu.async_copy(ones_v,
                                                 deg_sh.at[didx.at[g]],
                                                 dsem, add=True))
            for b in range(NBUF):
                g = g0 + b
                scat[(1 + with_deg) * b].wait()

                @pl.when(g + NBUF < gpc)
                def _():
                    gather(g + NBUF, b)

                if with_deg:
                    scat[(1 + with_deg) * b + 1].wait()
            return carry

        lax.fori_loop(0, gpc // NBUF, body, 0)

    plsc.subcore_barrier()
    pltpu.sync_copy(acc_sh.at[pl.ds(row0, RPT)],
                    acc_out.at[cid, pl.ds(row0, RPT)])
    if with_deg:
        pltpu.sync_copy(deg_sh.at[pl.ds(row0, RPT)],
                        deg_out.at[cid, pl.ds(row0, RPT)])


def _aggregate(h_split, src3, dst3, zrows, zdeg, n_groups, with_deg):
    mesh = plsc.VectorSubcoreMesh(core_axis_name="c", subcore_axis_name="s",
                                  num_cores=NC, num_subcores=NS)
    gpc = n_groups // NCHUNK
    k = pl.kernel(
        functools.partial(_agg_body, n_groups, with_deg),
        out_type=(jax.ShapeDtypeStruct((NC, R, CH), jnp.float32),
                  jax.ShapeDtypeStruct((NC, R), jnp.float32)),
        mesh=mesh,
        compiler_params=pltpu.CompilerParams(use_tc_tiling_on_sc=False),
        scratch_types=[
            pltpu.VMEM((gpc, GROUP), jnp.int32),
            pltpu.VMEM((gpc, GROUP), jnp.int32),
            pltpu.VMEM((NBUF, GROUP, CH), jnp.float32),
            pltpu.VMEM((GROUP,), jnp.float32),
            pltpu.VMEM_SHARED((R, CH), jnp.float32),
            pltpu.VMEM_SHARED((R, CH), jnp.float32),
            pltpu.VMEM_SHARED((R,), jnp.float32),
            pltpu.SemaphoreType.DMA((NBUF,)),
            pltpu.SemaphoreType.DMA((NBUF,)),
            pltpu.SemaphoreType.DMA,
        ],
    )
    return k(h_split, src3, dst3, zrows, zdeg)


def _conv_body(acc_ref, deg_ref, wc, bc, o_ref):
    s = jnp.concatenate([acc_ref[0], acc_ref[1]], axis=1)
    deg = jnp.maximum(deg_ref[0], 1.0)
    mean = s / deg[:, None]
    h = jax.nn.relu(
        jnp.dot(mean, wc[...], preferred_element_type=jnp.float32) + bc[...])
    o_ref[0] = h[:, :CH]
    o_ref[1] = h[:, CH:]


def _conv(acc, deg, Wc, bc):
    blk = 1280
    grid = R // blk
    full = lambda a: pl.BlockSpec(a.shape, lambda i: (0,) * a.ndim)
    return pl.pallas_call(
        _conv_body,
        grid=(grid,),
        in_specs=[
            pl.BlockSpec((NC, blk, CH), lambda i: (0, i, 0)),
            pl.BlockSpec((NC, blk), lambda i: (0, i)),
            full(Wc), full(bc),
        ],
        out_specs=pl.BlockSpec((NC, blk, CH), lambda i: (0, i, 0)),
        out_shape=jax.ShapeDtypeStruct((NC, R, CH), jnp.float32),
    )(acc, deg, Wc, bc)


def _conv_final_body(acc_ref, deg_ref, wc, bc, w5, b5, o_ref):
    s = jnp.concatenate([acc_ref[0], acc_ref[1]], axis=1)
    deg = jnp.maximum(deg_ref[0], 1.0)
    mean = s / deg[:, None]
    h = jax.nn.relu(
        jnp.dot(mean, wc[...], preferred_element_type=jnp.float32) + bc[...])
    o_ref[...] = jnp.dot(h, w5[...], preferred_element_type=jnp.float32) + b5[...]


def _conv_final(acc, deg, Wc, bc, W5p, b5p):
    blk = 1280
    grid = R // blk
    full = lambda a: pl.BlockSpec(a.shape, lambda i: (0,) * a.ndim)
    return pl.pallas_call(
        _conv_final_body,
        grid=(grid,),
        in_specs=[
            pl.BlockSpec((NC, blk, CH), lambda i: (0, i, 0)),
            pl.BlockSpec((NC, blk), lambda i: (0, i)),
            full(Wc), full(bc), full(W5p), full(b5p),
        ],
        out_specs=pl.BlockSpec((blk, 64), lambda i: (i, 0)),
        out_shape=jax.ShapeDtypeStruct((R, 64), jnp.float32),
    )(acc, deg, Wc, bc, W5p, b5p)


def kernel(x, edge_index, W1, b1, W2, b2, W3, b3, W4, b4,
           Wc1, bc1, Wc2, bc2, W5, b5):
    E = edge_index.shape[1]
    n_groups = -(-E // (NS * GROUP))
    n_groups = -(-n_groups // (NBUF * NCHUNK)) * (NBUF * NCHUNK)
    e_pad = NS * n_groups * GROUP

    src3 = (jnp.full((e_pad,), 0, jnp.int32).at[:E].set(edge_index[0])
            .reshape(NS, n_groups, GROUP))
    dst3 = (jnp.full((e_pad,), N_NODES, jnp.int32).at[:E].set(edge_index[1])
            .reshape(NS, n_groups, GROUP))
    x_pad = jnp.zeros((R, 5), jnp.float32).at[:N_NODES].set(x)
    zrows = jnp.zeros((R, CH), jnp.float32)
    zdeg = jnp.zeros((R,), jnp.float32)

    b1r, b2r = b1.reshape(1, -1), b2.reshape(1, -1)
    b3r, b4r = b3.reshape(1, -1), b4.reshape(1, -1)
    bc1r, bc2r = bc1.reshape(1, -1), bc2.reshape(1, -1)
    W5p = jnp.zeros((H, 64), jnp.float32).at[:, :60].set(W5)
    b5p = jnp.zeros((1, 64), jnp.float32).at[0, :60].set(b5)

    h = _mlp(x_pad, W1, b1r, W2, b2r, W3, b3r, W4, b4r)
    acc1, deg1 = _aggregate(h, src3, dst3, zrows, zdeg, n_groups, True)
    h2 = _conv(acc1, deg1, Wc1, bc1r)
    acc2, _ = _aggregate(h2, src3, dst3, zrows, zdeg, n_groups, False)
    out = _conv_final(acc2, deg1, Wc2, bc2r, W5p, b5p)
    return out[:N_NODES, :60]
